# Initial kernel scaffold; baseline (speedup 1.0000x reference)
import jax
import jax.numpy as jnp
from jax import lax
from jax.experimental import pallas as pl
from jax.experimental.pallas import tpu as pltpu
from jax.experimental.pallas import tpu_sc as plsc

N = 10000
E = 320000
D = 128
H = 128

NC = 2
NS = 16
NW = NC * NS
EW = E // NW
CH = 80
NCHUNK = EW // CH
NPAD = 10240
RW = NPAD // NS

F32 = jnp.float32


def _fill_1d(ref, n, val):
    v = jnp.full((16,), val, F32)

    def body(i, _):
        ref[pl.ds(i * 16, 16)] = v
        return 0

    lax.fori_loop(0, n // 16, body, 0)


def _fill_zero_2d(ref, rows):
    zv = jnp.zeros((16,), F32)

    def body(i, _):
        for j in range(D // 16):
            ref[i, pl.ds(j * 16, 16)] = zv
        return 0

    lax.fori_loop(0, rows, body, 0)


def _sc_degree(dst3, out, cnt, idx2, ones_v, zb):
    c = lax.axis_index("c")
    s = lax.axis_index("s")
    wid = s * NC + c

    _fill_1d(zb, RW, 0.0)
    _fill_1d(ones_v, CH, 1.0)

    pltpu.sync_copy(dst3.at[wid], idx2)
    pltpu.sync_copy(zb, cnt.at[pl.ds(s * RW, RW)])
    plsc.subcore_barrier()

    def body(k, _):
        pltpu.sync_copy(ones_v, cnt.at[idx2.at[k]], add=True)
        return 0

    lax.fori_loop(0, NCHUNK, body, 0)
    plsc.subcore_barrier()
    pltpu.sync_copy(cnt.at[pl.ds(s * RW, RW)], out.at[c, pl.ds(s * RW, RW)])


def _degree(dst3):
    return pl.kernel(
        _sc_degree,
        out_type=jax.ShapeDtypeStruct((NC, NPAD), F32),
        mesh=plsc.VectorSubcoreMesh(core_axis_name="c", subcore_axis_name="s"),
        scratch_types=[
            pltpu.VMEM_SHARED((NPAD,), F32),
            pltpu.VMEM((NCHUNK, CH), jnp.int32),
            pltpu.VMEM((CH,), F32),
            pltpu.VMEM((RW,), F32),
        ],
    )(dst3)


def _sc_scatter(g, src3, dst3, out, acc, src2, dst2, rows_a, rows_b,
                zb, sem_a, sem_b):
    c = lax.axis_index("c")
    s = lax.axis_index("s")
    wid = s * NC + c

    _fill_zero_2d(zb, 128)
    pltpu.sync_copy(src3.at[wid], src2)
    pltpu.sync_copy(dst3.at[wid], dst2)
    for q in range(RW // 128):
        pltpu.sync_copy(zb, acc.at[pl.ds(s * RW + q * 128, 128)])
    plsc.subcore_barrier()

    pltpu.async_copy(g.at[src2.at[0]], rows_a, sem_a)
    pltpu.async_copy(g.at[src2.at[1]], rows_b, sem_b)

    def body(i, _):
        k0 = 2 * i
        pltpu.make_async_copy(g.at[src2.at[k0]], rows_a, sem_a).wait()
        pltpu.sync_copy(rows_a, acc.at[dst2.at[k0]], add=True)
        pltpu.async_copy(g.at[src2.at[k0 + 2]], rows_a, sem_a)
        pltpu.make_async_copy(g.at[src2.at[k0 + 1]], rows_b, sem_b).wait()
        pltpu.sync_copy(rows_b, acc.at[dst2.at[k0 + 1]], add=True)

        @pl.when(i < (NCHUNK - 3) // 2)
        def _():
            pltpu.async_copy(g.at[src2.at[k0 + 3]], rows_b, sem_b)

        return 0

    lax.fori_loop(0, (NCHUNK - 1) // 2, body, 0)
    pltpu.make_async_copy(g.at[src2.at[NCHUNK - 1]], rows_a, sem_a).wait()
    pltpu.sync_copy(rows_a, acc.at[dst2.at[NCHUNK - 1]], add=True)

    plsc.subcore_barrier()
    pltpu.sync_copy(acc.at[pl.ds(s * RW, RW)], out.at[c, pl.ds(s * RW, RW)])


def _scatter_rows(g, src3, dst3):
    return pl.kernel(
        _sc_scatter,
        out_type=jax.ShapeDtypeStruct((NC, NPAD, D), F32),
        mesh=plsc.VectorSubcoreMesh(core_axis_name="c", subcore_axis_name="s"),
        scratch_types=[
            pltpu.VMEM_SHARED((NPAD, D), F32),
            pltpu.VMEM((NCHUNK, CH), jnp.int32),
            pltpu.VMEM((NCHUNK, CH), jnp.int32),
            pltpu.VMEM((CH, D), F32),
            pltpu.VMEM((CH, D), F32),
            pltpu.VMEM((128, D), F32),
            pltpu.SemaphoreType.DMA,
            pltpu.SemaphoreType.DMA,
        ],
    )(g, src3, dst3)


def _sc_score(a, b, ea, src, dst, out, a_v, b_v, sv, dv, ev, ov):
    c = lax.axis_index("c")
    s = lax.axis_index("s")
    wid = s * NC + c
    base = wid * EW

    pltpu.sync_copy(a, a_v)
    pltpu.sync_copy(b, b_v)
    pltpu.sync_copy(src.at[pl.ds(base, EW)], sv)
    pltpu.sync_copy(dst.at[pl.ds(base, EW)], dv)
    pltpu.sync_copy(ea.at[pl.ds(base, EW)], ev)

    def body(j, _):
        o = j * 16
        s16 = sv[pl.ds(o, 16)]
        d16 = dv[pl.ds(o, 16)]
        va = plsc.load_gather(a_v, [s16])
        vb = plsc.load_gather(b_v, [d16])
        ov[pl.ds(o, 16)] = va + vb + ev[pl.ds(o, 16)]
        return 0

    lax.fori_loop(0, EW // 16, body, 0)
    pltpu.sync_copy(ov, out.at[pl.ds(base, EW)])


def _score(a, b, ea, src, dst):
    return pl.kernel(
        _sc_score,
        out_type=jax.ShapeDtypeStruct((E,), F32),
        mesh=plsc.VectorSubcoreMesh(core_axis_name="c", subcore_axis_name="s"),
        scratch_types=[
            pltpu.VMEM((NPAD,), F32),
            pltpu.VMEM((NPAD,), F32),
            pltpu.VMEM((EW,), jnp.int32),
            pltpu.VMEM((EW,), jnp.int32),
            pltpu.VMEM((EW,), F32),
            pltpu.VMEM((EW,), F32),
        ],
    )(a, b, ea, src, dst)


NB = NPAD // 10
AR = N // 10


def _tc1_body(xb, w1b, d0b, d1b, atb, wmb, bpb, g1b, disb, eab):
    deg = d0b[...] + d1b[...] + 1.0
    dis = lax.rsqrt(deg)
    disb[...] = dis
    h = jnp.dot(xb[...], w1b[...], preferred_element_type=F32)
    g1b[...] = h * dis[:, None]
    eab[...] = jnp.dot(atb[...], wmb[...], preferred_element_type=F32) + bpb[0:1, :]


def _tc1(x_pad, w1, deg0, deg1, attr2d, wmat, bpb):
    return pl.pallas_call(
        _tc1_body,
        grid=(10,),
        in_specs=[
            pl.BlockSpec((NB, D), lambda i: (i, 0)),
            pl.BlockSpec((D, H), lambda i: (0, 0)),
            pl.BlockSpec((NB,), lambda i: (i,)),
            pl.BlockSpec((NB,), lambda i: (i,)),
            pl.BlockSpec((AR, 128), lambda i: (i, 0)),
            pl.BlockSpec((128, 128), lambda i: (0, 0)),
            pl.BlockSpec((8, 128), lambda i: (0, 0)),
        ],
        out_specs=[
            pl.BlockSpec((NB, H), lambda i: (i, 0)),
            pl.BlockSpec((NB,), lambda i: (i,)),
            pl.BlockSpec((AR, 128), lambda i: (i, 0)),
        ],
        out_shape=[
            jax.ShapeDtypeStruct((NPAD, H), F32),
            jax.ShapeDtypeStruct((NPAD,), F32),
            jax.ShapeDtypeStruct((N, 128), F32),
        ],
    )(x_pad, w1, deg0, deg1, attr2d, wmat, bpb)


def _tc2_body(a0b, a1b, g1b, db, b1b, w2b, g2b):
    dis = db[...]
    t = dis[:, None] * (a0b[...] + a1b[...] + g1b[...]) + b1b[0:1, :]
    t = jnp.maximum(t, 0.0)
    g2b[...] = jnp.dot(t, w2b[...], preferred_element_type=F32) * dis[:, None]


def _tc2(a0, a1, g1, dis, b1b, w2):
    return pl.pallas_call(
        _tc2_body,
        grid=(10,),
        in_specs=[
            pl.BlockSpec((NB, H), lambda i: (i, 0)),
            pl.BlockSpec((NB, H), lambda i: (i, 0)),
            pl.BlockSpec((NB, H), lambda i: (i, 0)),
            pl.BlockSpec((NB,), lambda i: (i,)),
            pl.BlockSpec((8, H), lambda i: (0, 0)),
            pl.BlockSpec((H, H), lambda i: (0, 0)),
        ],
        out_specs=pl.BlockSpec((NB, H), lambda i: (i, 0)),
        out_shape=jax.ShapeDtypeStruct((NPAD, H), F32),
    )(a0, a1, g1, dis, b1b, w2)


def _tc3_body(a0b, a1b, g2b, db, b2b, wabb, abb):
    dis = db[...]
    t = dis[:, None] * (a0b[...] + a1b[...] + g2b[...]) + b2b[0:1, :]
    t = jnp.maximum(t, 0.0)
    abb[...] = jnp.dot(t, wabb[...], preferred_element_type=F32)


def _tc3(a0, a1, g2, dis, b2b, wab):
    return pl.pallas_call(
        _tc3_body,
        grid=(10,),
        in_specs=[
            pl.BlockSpec((NB, H), lambda i: (i, 0)),
            pl.BlockSpec((NB, H), lambda i: (i, 0)),
            pl.BlockSpec((NB, H), lambda i: (i, 0)),
            pl.BlockSpec((NB,), lambda i: (i,)),
            pl.BlockSpec((8, H), lambda i: (0, 0)),
            pl.BlockSpec((H, 128), lambda i: (0, 0)),
        ],
        out_specs=pl.BlockSpec((NB, 128), lambda i: (i, 0)),
        out_shape=jax.ShapeDtypeStruct((NPAD, 128), F32),
    )(a0, a1, g2, dis, b2b, wab)


def kernel(x, edge_index, edge_attr, W1, b1, W2, b2, Wp, bp):
    src = edge_index[0].astype(jnp.int32)
    dst = edge_index[1].astype(jnp.int32)
    src3 = src.reshape(NW, NCHUNK, CH)
    dst3 = dst.reshape(NW, NCHUNK, CH)

    x_pad = jnp.pad(x, ((0, NPAD - N), (0, 0)))
    attr2d = edge_attr.reshape(N, 128)

    wp_e = Wp[2 * H:, 0]
    eye32 = jnp.eye(32, dtype=F32)
    wmat = jnp.pad(jnp.kron(eye32, wp_e[:, None]), ((0, 0), (0, 96)))

    wab = jnp.zeros((H, 128), F32)
    wab = wab.at[:, 0].set(Wp[:H, 0]).at[:, 1].set(Wp[H:2 * H, 0])

    bpb = jnp.broadcast_to(bp.reshape(1, 1), (8, 128)).astype(F32)
    b1b = jnp.broadcast_to(b1[None, :], (8, H))
    b2b = jnp.broadcast_to(b2[None, :], (8, H))

    degp = _degree(dst3)
    g1, dis, ea2d = _tc1(x_pad, W1, degp[0], degp[1], attr2d, wmat, bpb)
    a1 = _scatter_rows(g1, src3, dst3)
    g2 = _tc2(a1[0], a1[1], g1, dis, b1b, W2)
    a2 = _scatter_rows(g2, src3, dst3)
    abm = _tc3(a2[0], a2[1], g2, dis, b2b, wab)
    ea = ea2d[:, :32].reshape(E)
    scores = _score(abm[:, 0], abm[:, 1], ea, src, dst)
    return scores.reshape(E, 1)


# speedup vs baseline: 9.0013x; 9.0013x over previous
import jax
import jax.numpy as jnp
from jax import lax
from jax.experimental import pallas as pl
from jax.experimental.pallas import tpu as pltpu
from jax.experimental.pallas import tpu_sc as plsc

N = 10000
E = 320000
D = 128
H = 128

NC = 2
NS = 16
NW = NC * NS
CH = 128
NCHUNK = 80
HC = NCHUNK // 2
EPAD = NW * NCHUNK * CH
EW = EPAD // NW
NPAD = 10240
RW = NPAD // NS

F32 = jnp.float32


def _fill_1d(ref, n, val):
    v = jnp.full((16,), val, F32)

    def body(i, _):
        ref[pl.ds(i * 16, 16)] = v
        return 0

    lax.fori_loop(0, n // 16, body, 0)


def _fill_zero_2d(ref, rows):
    zv = jnp.zeros((16,), F32)

    def body(i, _):
        for j in range(D // 16):
            ref[i, pl.ds(j * 16, 16)] = zv
        return 0

    lax.fori_loop(0, rows, body, 0)


def _sc_degree(dst3, out, cnt, idx2, ones_v, zb):
    c = lax.axis_index("c")
    s = lax.axis_index("s")
    wid = s * NC + c

    _fill_1d(zb, RW, 0.0)
    _fill_1d(ones_v, CH, 1.0)

    pltpu.sync_copy(dst3.at[wid], idx2)
    pltpu.sync_copy(zb, cnt.at[pl.ds(s * RW, RW)])
    plsc.subcore_barrier()

    def body(k, _):
        pltpu.sync_copy(ones_v, cnt.at[idx2.at[k]], add=True)
        return 0

    lax.fori_loop(0, NCHUNK, body, 0)
    plsc.subcore_barrier()
    pltpu.sync_copy(cnt.at[pl.ds(s * RW, RW)], out.at[c, pl.ds(s * RW, RW)])


def _degree(dst3):
    return pl.kernel(
        _sc_degree,
        out_type=jax.ShapeDtypeStruct((NC, NPAD), F32),
        mesh=plsc.VectorSubcoreMesh(core_axis_name="c", subcore_axis_name="s"),
        scratch_types=[
            pltpu.VMEM_SHARED((NPAD,), F32),
            pltpu.VMEM((NCHUNK, CH), jnp.int32),
            pltpu.VMEM((CH,), F32),
            pltpu.VMEM((RW,), F32),
        ],
    )(dst3)


def _sc_scatter(g, src3, dst3, out0, out1, acc, srch, dst2, rows_a, rows_b,
                sem_ga, sem_gb):
    c = lax.axis_index("c")
    s = lax.axis_index("s")
    wid = s * NC + c

    _fill_zero_2d(rows_a, CH)
    pltpu.sync_copy(dst3.at[wid], dst2)
    for q in range(RW // CH):
        pltpu.sync_copy(rows_a, acc.at[pl.ds(s * RW + q * CH, CH)])
    plsc.subcore_barrier()

    def do_half(h):
        hb = h * HC
        pltpu.sync_copy(src3.at[wid, pl.ds(hb, HC)], srch)
        pltpu.async_copy(g.at[srch.at[0]], rows_a, sem_ga)
        pltpu.async_copy(g.at[srch.at[1]], rows_b, sem_gb)

        def body(i, _):
            k0 = 2 * i
            pltpu.make_async_copy(g.at[srch.at[k0]], rows_a, sem_ga).wait()
            pltpu.sync_copy(rows_a, acc.at[dst2.at[hb + k0]], add=True)

            @pl.when(i < HC // 2 - 1)
            def _():
                pltpu.async_copy(g.at[srch.at[k0 + 2]], rows_a, sem_ga)

            pltpu.make_async_copy(g.at[srch.at[k0 + 1]], rows_b, sem_gb).wait()
            pltpu.sync_copy(rows_b, acc.at[dst2.at[hb + k0 + 1]], add=True)

            @pl.when(i < HC // 2 - 1)
            def _():
                pltpu.async_copy(g.at[srch.at[k0 + 3]], rows_b, sem_gb)

            return 0

        lax.fori_loop(0, HC // 2, body, 0)

    do_half(0)
    do_half(1)

    plsc.subcore_barrier()

    @pl.when(c == 0)
    def _():
        pltpu.sync_copy(acc.at[pl.ds(s * RW, RW)], out0.at[pl.ds(s * RW, RW)])

    @pl.when(c == 1)
    def _():
        pltpu.sync_copy(acc.at[pl.ds(s * RW, RW)], out1.at[pl.ds(s * RW, RW)])


def _scatter_rows(g, src3, dst3):
    return pl.kernel(
        _sc_scatter,
        out_type=(jax.ShapeDtypeStruct((NPAD, D), F32),
                  jax.ShapeDtypeStruct((NPAD, D), F32)),
        mesh=plsc.VectorSubcoreMesh(core_axis_name="c", subcore_axis_name="s"),
        scratch_types=[
            pltpu.VMEM_SHARED((NPAD, D), F32),
            pltpu.VMEM((HC, CH), jnp.int32),
            pltpu.VMEM((NCHUNK, CH), jnp.int32),
            pltpu.VMEM((CH, D), F32),
            pltpu.VMEM((CH, D), F32),
            pltpu.SemaphoreType.DMA,
            pltpu.SemaphoreType.DMA,
        ],
    )(g, src3, dst3)


def _sc_score(a, b, ea, src, dst, out, a_v, b_v, sv, dv, ev, ov):
    c = lax.axis_index("c")
    s = lax.axis_index("s")
    wid = s * NC + c
    base = wid * EW

    pltpu.sync_copy(a, a_v)
    pltpu.sync_copy(b, b_v)
    pltpu.sync_copy(src.at[pl.ds(base, EW)], sv)
    pltpu.sync_copy(dst.at[pl.ds(base, EW)], dv)
    pltpu.sync_copy(ea.at[pl.ds(base, EW)], ev)

    def body(j, _):
        o = j * 16
        s16 = sv[pl.ds(o, 16)]
        d16 = dv[pl.ds(o, 16)]
        va = plsc.load_gather(a_v, [s16])
        vb = plsc.load_gather(b_v, [d16])
        ov[pl.ds(o, 16)] = va + vb + ev[pl.ds(o, 16)]
        return 0

    lax.fori_loop(0, EW // 16, body, 0)
    pltpu.sync_copy(ov, out.at[pl.ds(base, EW)])


def _score(a, b, ea, src, dst):
    return pl.kernel(
        _sc_score,
        out_type=jax.ShapeDtypeStruct((EPAD,), F32),
        mesh=plsc.VectorSubcoreMesh(core_axis_name="c", subcore_axis_name="s"),
        compiler_params=pltpu.CompilerParams(needs_layout_passes=False),
        scratch_types=[
            pltpu.VMEM((NPAD,), F32),
            pltpu.VMEM((NPAD,), F32),
            pltpu.VMEM((EW,), jnp.int32),
            pltpu.VMEM((EW,), jnp.int32),
            pltpu.VMEM((EW,), F32),
            pltpu.VMEM((EW,), F32),
        ],
    )(a, b, ea, src, dst)


NB = NPAD // 10
AR = N // 10


def _tc1_body(xb, w1b, d0b, d1b, atb, wmb, bpb, g1b, disb, eab):
    deg = d0b[...] + d1b[...] + 1.0
    dis = lax.rsqrt(deg)
    disb[...] = dis
    h = jnp.dot(xb[...], w1b[...], preferred_element_type=F32)
    g1b[...] = h * dis[:, None]
    at2 = jnp.reshape(atb[...], (AR, 128))
    eab[...] = jnp.dot(at2, wmb[...], preferred_element_type=F32) + bpb[0:1, :]


def _tc1(x_pad, w1, deg0, deg1, attr1, wmat, bpb):
    return pl.pallas_call(
        _tc1_body,
        grid=(10,),
        in_specs=[
            pl.BlockSpec((NB, D), lambda i: (i, 0)),
            pl.BlockSpec((D, H), lambda i: (0, 0)),
            pl.BlockSpec((NB,), lambda i: (i,)),
            pl.BlockSpec((NB,), lambda i: (i,)),
            pl.BlockSpec((AR * 128,), lambda i: (i,)),
            pl.BlockSpec((128, 128), lambda i: (0, 0)),
            pl.BlockSpec((8, 128), lambda i: (0, 0)),
        ],
        out_specs=[
            pl.BlockSpec((NB, H), lambda i: (i, 0)),
            pl.BlockSpec((NB,), lambda i: (i,)),
            pl.BlockSpec((AR, 128), lambda i: (i, 0)),
        ],
        out_shape=[
            jax.ShapeDtypeStruct((NPAD, H), F32),
            jax.ShapeDtypeStruct((NPAD,), F32),
            jax.ShapeDtypeStruct((N, 128), F32),
        ],
    )(x_pad, w1, deg0, deg1, attr1, wmat, bpb)


def _tc2_body(a0b, a1b, g1b, db, b1b, w2b, g2b):
    dis = db[...]
    t = dis[:, None] * (a0b[...] + a1b[...] + g1b[...]) + b1b[0:1, :]
    t = jnp.maximum(t, 0.0)
    g2b[...] = jnp.dot(t, w2b[...], preferred_element_type=F32) * dis[:, None]


def _tc2(a0, a1, g1, dis, b1b, w2):
    return pl.pallas_call(
        _tc2_body,
        grid=(10,),
        in_specs=[
            pl.BlockSpec((NB, H), lambda i: (i, 0)),
            pl.BlockSpec((NB, H), lambda i: (i, 0)),
            pl.BlockSpec((NB, H), lambda i: (i, 0)),
            pl.BlockSpec((NB,), lambda i: (i,)),
            pl.BlockSpec((8, H), lambda i: (0, 0)),
            pl.BlockSpec((H, H), lambda i: (0, 0)),
        ],
        out_specs=pl.BlockSpec((NB, H), lambda i: (i, 0)),
        out_shape=jax.ShapeDtypeStruct((NPAD, H), F32),
    )(a0, a1, g1, dis, b1b, w2)


def _tc3_body(a0b, a1b, g2b, db, b2b, wabb, abb):
    dis = db[...]
    t = dis[:, None] * (a0b[...] + a1b[...] + g2b[...]) + b2b[0:1, :]
    t = jnp.maximum(t, 0.0)
    abb[...] = jnp.dot(t, wabb[...], preferred_element_type=F32)


def _tc3(a0, a1, g2, dis, b2b, wab):
    return pl.pallas_call(
        _tc3_body,
        grid=(10,),
        in_specs=[
            pl.BlockSpec((NB, H), lambda i: (i, 0)),
            pl.BlockSpec((NB, H), lambda i: (i, 0)),
            pl.BlockSpec((NB, H), lambda i: (i, 0)),
            pl.BlockSpec((NB,), lambda i: (i,)),
            pl.BlockSpec((8, H), lambda i: (0, 0)),
            pl.BlockSpec((H, 128), lambda i: (0, 0)),
        ],
        out_specs=pl.BlockSpec((NB, 128), lambda i: (i, 0)),
        out_shape=jax.ShapeDtypeStruct((NPAD, 128), F32),
    )(a0, a1, g2, dis, b2b, wab)


def kernel(x, edge_index, edge_attr, W1, b1, W2, b2, Wp, bp):
    ei = edge_index.astype(jnp.int32)
    pad_n = EPAD - E
    src = jnp.pad(ei[0], (0, pad_n), constant_values=NPAD - 1)
    dst = jnp.pad(ei[1], (0, pad_n), constant_values=NPAD - 1)
    src3 = src.reshape(NW, NCHUNK, CH)
    dst3 = dst.reshape(NW, NCHUNK, CH)

    x_pad = jnp.pad(x, ((0, NPAD - N), (0, 0)))
    attr1 = edge_attr.reshape(E * 4)

    wp_e = Wp[2 * H:, 0]
    eye32 = jnp.eye(32, dtype=F32)
    wmat = jnp.pad(jnp.kron(eye32, wp_e[:, None]), ((0, 0), (0, 96)))

    wab = jnp.zeros((H, 128), F32)
    wab = wab.at[:, 0].set(Wp[:H, 0]).at[:, 1].set(Wp[H:2 * H, 0])

    bpb = jnp.broadcast_to(bp.reshape(1, 1), (8, 128)).astype(F32)
    b1b = jnp.broadcast_to(b1[None, :], (8, H))
    b2b = jnp.broadcast_to(b2[None, :], (8, H))

    degp = _degree(dst3)
    g1, dis, ea2d = _tc1(x_pad, W1, degp[0], degp[1], attr1, wmat, bpb)
    a10, a11 = _scatter_rows(g1, src3, dst3)
    g2 = _tc2(a10, a11, g1, dis, b1b, W2)
    a20, a21 = _scatter_rows(g2, src3, dst3)
    abm = _tc3(a20, a21, g2, dis, b2b, wab)
    ea = jnp.pad(ea2d[:, :32].reshape(E), (0, pad_n))
    scores = _score(abm[:, 0], abm[:, 1], ea, src, dst)
    return scores[:E].reshape(E, 1)


# speedup vs baseline: 20.4014x; 2.2665x over previous
import jax
import jax.numpy as jnp
from jax import lax
from jax.experimental import pallas as pl
from jax.experimental.pallas import tpu as pltpu
from jax.experimental.pallas import tpu_sc as plsc

N = 10000
E = 320000
D = 128
H = 128

NC = 2
NS = 16
NW = NC * NS
EW = E // NW
CH = 80
NCHUNK = EW // CH
NPAD = 10240
RW = NPAD // NS

F32 = jnp.float32


def _fill_1d(ref, n, val):
    v = jnp.full((16,), val, F32)

    def body(i, _):
        ref[pl.ds(i * 16, 16)] = v
        return 0

    lax.fori_loop(0, n // 16, body, 0)


def _fill_zero_2d(ref, rows):
    zv = jnp.zeros((16,), F32)

    def body(i, _):
        for j in range(D // 16):
            ref[i, pl.ds(j * 16, 16)] = zv
        return 0

    lax.fori_loop(0, rows, body, 0)


def _sc_degree(dst3, out, cnt, idx2, ones_v, zb):
    c = lax.axis_index("c")
    s = lax.axis_index("s")
    wid = s * NC + c

    _fill_1d(zb, RW, 0.0)
    _fill_1d(ones_v, CH, 1.0)

    pltpu.sync_copy(dst3.at[wid], idx2)
    pltpu.sync_copy(zb, cnt.at[pl.ds(s * RW, RW)])
    plsc.subcore_barrier()

    def body(k, _):
        pltpu.sync_copy(ones_v, cnt.at[idx2.at[k]], add=True)
        return 0

    lax.fori_loop(0, NCHUNK, body, 0)
    plsc.subcore_barrier()
    pltpu.sync_copy(cnt.at[pl.ds(s * RW, RW)], out.at[c, pl.ds(s * RW, RW)])


def _degree(dst3):
    return pl.kernel(
        _sc_degree,
        out_type=jax.ShapeDtypeStruct((NC, NPAD), F32),
        mesh=plsc.VectorSubcoreMesh(core_axis_name="c", subcore_axis_name="s"),
        scratch_types=[
            pltpu.VMEM_SHARED((NPAD,), F32),
            pltpu.VMEM((NCHUNK, CH), jnp.int32),
            pltpu.VMEM((CH,), F32),
            pltpu.VMEM((RW,), F32),
        ],
    )(dst3)


def _sc_scatter(g, src, dst3, out0, out1, acc, src1, dst2, rows_a, rows_b,
                sem_a, sem_b):
    c = lax.axis_index("c")
    s = lax.axis_index("s")
    wid = s * NC + c

    _fill_zero_2d(rows_a, CH)
    pltpu.sync_copy(src.at[pl.ds(wid * EW, EW)], src1)
    pltpu.sync_copy(dst3.at[wid], dst2)
    for q in range(RW // CH):
        pltpu.sync_copy(rows_a, acc.at[pl.ds(s * RW + q * CH, CH)])
    plsc.subcore_barrier()

    def idx(k):
        return src1.at[pl.ds(k * CH, CH)]

    pltpu.async_copy(g.at[idx(0)], rows_a, sem_a)
    pltpu.async_copy(g.at[idx(1)], rows_b, sem_b)

    def body(i, _):
        k0 = 2 * i
        pltpu.make_async_copy(g.at[idx(k0)], rows_a, sem_a).wait()
        pltpu.sync_copy(rows_a, acc.at[dst2.at[k0]], add=True)
        pltpu.async_copy(g.at[idx(k0 + 2)], rows_a, sem_a)
        pltpu.make_async_copy(g.at[idx(k0 + 1)], rows_b, sem_b).wait()
        pltpu.sync_copy(rows_b, acc.at[dst2.at[k0 + 1]], add=True)

        @pl.when(i < (NCHUNK - 3) // 2)
        def _():
            pltpu.async_copy(g.at[idx(k0 + 3)], rows_b, sem_b)

        return 0

    lax.fori_loop(0, (NCHUNK - 1) // 2, body, 0)
    pltpu.make_async_copy(g.at[idx(NCHUNK - 1)], rows_a, sem_a).wait()
    pltpu.sync_copy(rows_a, acc.at[dst2.at[NCHUNK - 1]], add=True)

    plsc.subcore_barrier()

    @pl.when(c == 0)
    def _():
        pltpu.sync_copy(acc.at[pl.ds(s * RW, RW)], out0.at[pl.ds(s * RW, RW)])

    @pl.when(c == 1)
    def _():
        pltpu.sync_copy(acc.at[pl.ds(s * RW, RW)], out1.at[pl.ds(s * RW, RW)])


def _scatter_rows(g, src, dst3):
    return pl.kernel(
        _sc_scatter,
        out_type=(jax.ShapeDtypeStruct((NPAD, D), F32),
                  jax.ShapeDtypeStruct((NPAD, D), F32)),
        mesh=plsc.VectorSubcoreMesh(core_axis_name="c", subcore_axis_name="s"),
        scratch_types=[
            pltpu.VMEM_SHARED((NPAD, D), F32),
            pltpu.VMEM((EW,), jnp.int32),
            pltpu.VMEM((NCHUNK, CH), jnp.int32),
            pltpu.VMEM((CH, D), F32),
            pltpu.VMEM((CH, D), F32),
            pltpu.SemaphoreType.DMA,
            pltpu.SemaphoreType.DMA,
        ],
    )(g, src, dst3)


def _sc_score(a, b, ea, src, dst, out, a_v, b_v, sv, dv, ev, ov):
    c = lax.axis_index("c")
    s = lax.axis_index("s")
    wid = s * NC + c
    base = wid * EW

    pltpu.sync_copy(a, a_v)
    pltpu.sync_copy(b, b_v)
    pltpu.sync_copy(src.at[pl.ds(base, EW)], sv)
    pltpu.sync_copy(dst.at[pl.ds(base, EW)], dv)
    pltpu.sync_copy(ea.at[pl.ds(base, EW)], ev)

    def body(j, _):
        o = j * 16
        s16 = sv[pl.ds(o, 16)]
        d16 = dv[pl.ds(o, 16)]
        va = plsc.load_gather(a_v, [s16])
        vb = plsc.load_gather(b_v, [d16])
        ov[pl.ds(o, 16)] = va + vb + ev[pl.ds(o, 16)]
        return 0

    lax.fori_loop(0, EW // 16, body, 0)
    pltpu.sync_copy(ov, out.at[pl.ds(base, EW)])


def _score(a, b, ea, src, dst):
    return pl.kernel(
        _sc_score,
        out_type=jax.ShapeDtypeStruct((E,), F32),
        mesh=plsc.VectorSubcoreMesh(core_axis_name="c", subcore_axis_name="s"),
        compiler_params=pltpu.CompilerParams(needs_layout_passes=False),
        scratch_types=[
            pltpu.VMEM((NPAD,), F32),
            pltpu.VMEM((NPAD,), F32),
            pltpu.VMEM((EW,), jnp.int32),
            pltpu.VMEM((EW,), jnp.int32),
            pltpu.VMEM((EW,), F32),
            pltpu.VMEM((EW,), F32),
        ],
    )(a, b, ea, src, dst)


NB = NPAD // 10
EB = E // 10


def _tc0_body(eib, atb, wpb, bpb, srcb, dstb, eab):
    ei = eib[...]
    srcb[...] = ei[0, :].reshape(1, 1, EB)
    dstb[...] = ei[1, :].reshape(1, 1, EB)
    at4 = atb[...]
    ea = jnp.sum(at4 * wpb[0:1, 0:4], axis=1) + bpb[0, 0]
    eab[...] = ea.reshape(1, 1, EB)


def _tc0(edge_index, edge_attr, wpeb, bpb):
    return pl.pallas_call(
        _tc0_body,
        grid=(10,),
        in_specs=[
            pl.BlockSpec((2, EB), lambda i: (0, i)),
            pl.BlockSpec((EB, 4), lambda i: (i, 0)),
            pl.BlockSpec((8, 128), lambda i: (0, 0)),
            pl.BlockSpec((8, 128), lambda i: (0, 0)),
        ],
        out_specs=[
            pl.BlockSpec((1, 1, EB), lambda i: (i, 0, 0)),
            pl.BlockSpec((1, 1, EB), lambda i: (i, 0, 0)),
            pl.BlockSpec((1, 1, EB), lambda i: (i, 0, 0)),
        ],
        out_shape=[
            jax.ShapeDtypeStruct((10, 1, EB), jnp.int32),
            jax.ShapeDtypeStruct((10, 1, EB), jnp.int32),
            jax.ShapeDtypeStruct((10, 1, EB), F32),
        ],
    )(edge_index, edge_attr, wpeb, bpb)


def _tc1_body(xb, w1b, d0b, d1b, g1b, disb):
    deg = d0b[...] + d1b[...] + 1.0
    dis = lax.rsqrt(deg)
    disb[...] = dis
    h = jnp.dot(xb[...], w1b[...], preferred_element_type=F32)
    g1b[...] = h * dis[:, None]


def _tc1(x_pad, w1, deg0, deg1):
    return pl.pallas_call(
        _tc1_body,
        grid=(10,),
        in_specs=[
            pl.BlockSpec((NB, D), lambda i: (i, 0)),
            pl.BlockSpec((D, H), lambda i: (0, 0)),
            pl.BlockSpec((NB,), lambda i: (i,)),
            pl.BlockSpec((NB,), lambda i: (i,)),
        ],
        out_specs=[
            pl.BlockSpec((NB, H), lambda i: (i, 0)),
            pl.BlockSpec((NB,), lambda i: (i,)),
        ],
        out_shape=[
            jax.ShapeDtypeStruct((NPAD, H), F32),
            jax.ShapeDtypeStruct((NPAD,), F32),
        ],
    )(x_pad, w1, deg0, deg1)


def _tc2_body(a0b, a1b, g1b, db, b1b, w2b, g2b):
    dis = db[...]
    t = dis[:, None] * (a0b[...] + a1b[...] + g1b[...]) + b1b[0:1, :]
    t = jnp.maximum(t, 0.0)
    g2b[...] = jnp.dot(t, w2b[...], preferred_element_type=F32) * dis[:, None]


def _tc2(a0, a1, g1, dis, b1b, w2):
    return pl.pallas_call(
        _tc2_body,
        grid=(10,),
        in_specs=[
            pl.BlockSpec((NB, H), lambda i: (i, 0)),
            pl.BlockSpec((NB, H), lambda i: (i, 0)),
            pl.BlockSpec((NB, H), lambda i: (i, 0)),
            pl.BlockSpec((NB,), lambda i: (i,)),
            pl.BlockSpec((8, H), lambda i: (0, 0)),
            pl.BlockSpec((H, H), lambda i: (0, 0)),
        ],
        out_specs=pl.BlockSpec((NB, H), lambda i: (i, 0)),
        out_shape=jax.ShapeDtypeStruct((NPAD, H), F32),
    )(a0, a1, g1, dis, b1b, w2)


def _tc3_body(a0b, a1b, g2b, db, b2b, wabb, abb):
    dis = db[...]
    t = dis[:, None] * (a0b[...] + a1b[...] + g2b[...]) + b2b[0:1, :]
    t = jnp.maximum(t, 0.0)
    abb[...] = jnp.dot(t, wabb[...], preferred_element_type=F32)


def _tc3(a0, a1, g2, dis, b2b, wab):
    return pl.pallas_call(
        _tc3_body,
        grid=(10,),
        in_specs=[
            pl.BlockSpec((NB, H), lambda i: (i, 0)),
            pl.BlockSpec((NB, H), lambda i: (i, 0)),
            pl.BlockSpec((NB, H), lambda i: (i, 0)),
            pl.BlockSpec((NB,), lambda i: (i,)),
            pl.BlockSpec((8, H), lambda i: (0, 0)),
            pl.BlockSpec((H, 128), lambda i: (0, 0)),
        ],
        out_specs=pl.BlockSpec((NB, 128), lambda i: (i, 0)),
        out_shape=jax.ShapeDtypeStruct((NPAD, 128), F32),
    )(a0, a1, g2, dis, b2b, wab)


def kernel(x, edge_index, edge_attr, W1, b1, W2, b2, Wp, bp):
    ei = edge_index.astype(jnp.int32)

    x_pad = jnp.pad(x, ((0, NPAD - N), (0, 0)))

    wab = jnp.zeros((H, 128), F32)
    wab = wab.at[:, 0].set(Wp[:H, 0]).at[:, 1].set(Wp[H:2 * H, 0])

    wpeb = jnp.pad(Wp[2 * H:, 0][None, :], ((0, 7), (0, 124)))
    bpb = jnp.broadcast_to(bp.reshape(1, 1), (8, 128)).astype(F32)
    b1b = jnp.broadcast_to(b1[None, :], (8, H))
    b2b = jnp.broadcast_to(b2[None, :], (8, H))

    src_u, dst_u, ea_u = _tc0(ei, edge_attr, wpeb, bpb)
    src = src_u.reshape(E)
    dst = dst_u.reshape(E)
    ea = ea_u.reshape(E)
    src3 = src.reshape(NW, NCHUNK, CH)
    dst3 = dst.reshape(NW, NCHUNK, CH)

    degp = _degree(dst3)
    g1, dis = _tc1(x_pad, W1, degp[0], degp[1])
    a10, a11 = _scatter_rows(g1, src, dst3)
    g2 = _tc2(a10, a11, g1, dis, b1b, W2)
    a20, a21 = _scatter_rows(g2, src, dst3)
    abm = _tc3(a20, a21, g2, dis, b2b, wab)
    scores = _score(abm[:, 0], abm[:, 1], ea, src, dst)
    return scores.reshape(E, 1)


# speedup vs baseline: 24.0389x; 1.1783x over previous
import jax
import jax.numpy as jnp
from jax import lax
from jax.experimental import pallas as pl
from jax.experimental.pallas import tpu as pltpu
from jax.experimental.pallas import tpu_sc as plsc

N = 10000
E = 320000
D = 128
H = 128

NC = 2
NS = 16
NW = NC * NS
EW = E // NW
CH = 80
NCHUNK = EW // CH
NPAD = 10240
RW = NPAD // NS

F32 = jnp.float32


def _fill_1d(ref, n, val):
    v = jnp.full((16,), val, F32)

    def body(i, _):
        ref[pl.ds(i * 16, 16)] = v
        return 0

    lax.fori_loop(0, n // 16, body, 0)


def _fill_zero_2d(ref, rows):
    zv = jnp.zeros((16,), F32)

    def body(i, _):
        for j in range(D // 16):
            ref[i, pl.ds(j * 16, 16)] = zv
        return 0

    lax.fori_loop(0, rows, body, 0)


def _sc_degree(dst3, out, cnt, idx2, ones_v, zb):
    c = lax.axis_index("c")
    s = lax.axis_index("s")
    wid = s * NC + c

    _fill_1d(zb, RW, 0.0)
    _fill_1d(ones_v, CH, 1.0)

    pltpu.sync_copy(dst3.at[wid], idx2)
    pltpu.sync_copy(zb, cnt.at[pl.ds(s * RW, RW)])
    plsc.subcore_barrier()

    def body(k, _):
        pltpu.sync_copy(ones_v, cnt.at[idx2.at[k]], add=True)
        return 0

    lax.fori_loop(0, NCHUNK, body, 0)
    plsc.subcore_barrier()
    pltpu.sync_copy(cnt.at[pl.ds(s * RW, RW)], out.at[c, pl.ds(s * RW, RW)])


def _degree(dst3):
    return pl.kernel(
        _sc_degree,
        out_type=jax.ShapeDtypeStruct((NC, NPAD), F32),
        mesh=plsc.VectorSubcoreMesh(core_axis_name="c", subcore_axis_name="s"),
        scratch_types=[
            pltpu.VMEM_SHARED((NPAD,), F32),
            pltpu.VMEM((NCHUNK, CH), jnp.int32),
            pltpu.VMEM((CH,), F32),
            pltpu.VMEM((RW,), F32),
        ],
    )(dst3)


def _sc_scatter(g, src, dst3, out0, out1, acc, src1, dst2, rows_a, rows_b,
                sem_a, sem_b):
    c = lax.axis_index("c")
    s = lax.axis_index("s")
    wid = s * NC + c

    _fill_zero_2d(rows_a, CH)
    pltpu.sync_copy(src.at[pl.ds(wid * EW, EW)], src1)
    pltpu.sync_copy(dst3.at[wid], dst2)
    for q in range(RW // CH):
        pltpu.sync_copy(rows_a, acc.at[pl.ds(s * RW + q * CH, CH)])
    plsc.subcore_barrier()

    def idx(k):
        return src1.at[pl.ds(k * CH, CH)]

    pltpu.async_copy(g.at[idx(0)], rows_a, sem_a)
    pltpu.async_copy(g.at[idx(1)], rows_b, sem_b)

    def body(i, _):
        k0 = 2 * i
        pltpu.make_async_copy(g.at[idx(k0)], rows_a, sem_a).wait()
        pltpu.sync_copy(rows_a, acc.at[dst2.at[k0]], add=True)
        pltpu.async_copy(g.at[idx(k0 + 2)], rows_a, sem_a)
        pltpu.make_async_copy(g.at[idx(k0 + 1)], rows_b, sem_b).wait()
        pltpu.sync_copy(rows_b, acc.at[dst2.at[k0 + 1]], add=True)

        @pl.when(i < (NCHUNK - 3) // 2)
        def _():
            pltpu.async_copy(g.at[idx(k0 + 3)], rows_b, sem_b)

        return 0

    lax.fori_loop(0, (NCHUNK - 1) // 2, body, 0)
    pltpu.make_async_copy(g.at[idx(NCHUNK - 1)], rows_a, sem_a).wait()
    pltpu.sync_copy(rows_a, acc.at[dst2.at[NCHUNK - 1]], add=True)

    plsc.subcore_barrier()

    @pl.when(c == 0)
    def _():
        pltpu.sync_copy(acc.at[pl.ds(s * RW, RW)], out0.at[pl.ds(s * RW, RW)])

    @pl.when(c == 1)
    def _():
        pltpu.sync_copy(acc.at[pl.ds(s * RW, RW)], out1.at[pl.ds(s * RW, RW)])


def _scatter_rows(g, src, dst3):
    return pl.kernel(
        _sc_scatter,
        out_type=(jax.ShapeDtypeStruct((NPAD, D), F32),
                  jax.ShapeDtypeStruct((NPAD, D), F32)),
        mesh=plsc.VectorSubcoreMesh(core_axis_name="c", subcore_axis_name="s"),
        scratch_types=[
            pltpu.VMEM_SHARED((NPAD, D), F32),
            pltpu.VMEM((EW,), jnp.int32),
            pltpu.VMEM((NCHUNK, CH), jnp.int32),
            pltpu.VMEM((CH, D), F32),
            pltpu.VMEM((CH, D), F32),
            pltpu.SemaphoreType.DMA,
            pltpu.SemaphoreType.DMA,
        ],
    )(g, src, dst3)


def _sc_score(a, b, ea, src, dst, out, a_v, b_v, sv, dv, ev, ov):
    c = lax.axis_index("c")
    s = lax.axis_index("s")
    wid = s * NC + c
    base = wid * EW

    pltpu.sync_copy(a, a_v)
    pltpu.sync_copy(b, b_v)
    pltpu.sync_copy(src.at[pl.ds(base, EW)], sv)
    pltpu.sync_copy(dst.at[pl.ds(base, EW)], dv)
    pltpu.sync_copy(ea.at[pl.ds(base, EW)], ev)

    def body(j, _):
        o = j * 16
        s16 = sv[pl.ds(o, 16)]
        d16 = dv[pl.ds(o, 16)]
        va = plsc.load_gather(a_v, [s16])
        vb = plsc.load_gather(b_v, [d16])
        ov[pl.ds(o, 16)] = va + vb + ev[pl.ds(o, 16)]
        return 0

    lax.fori_loop(0, EW // 16, body, 0)
    pltpu.sync_copy(ov, out.at[pl.ds(base, EW)])


def _score(a, b, ea, src, dst):
    return pl.kernel(
        _sc_score,
        out_type=jax.ShapeDtypeStruct((E,), F32),
        mesh=plsc.VectorSubcoreMesh(core_axis_name="c", subcore_axis_name="s"),
        compiler_params=pltpu.CompilerParams(needs_layout_passes=False),
        scratch_types=[
            pltpu.VMEM((NPAD,), F32),
            pltpu.VMEM((NPAD,), F32),
            pltpu.VMEM((EW,), jnp.int32),
            pltpu.VMEM((EW,), jnp.int32),
            pltpu.VMEM((EW,), F32),
            pltpu.VMEM((EW,), F32),
        ],
    )(a, b, ea, src, dst)


NB = NPAD // 10
AR = N // 10


def _tc1_body(xb, w1b, d0b, d1b, g1b, disb):
    deg = d0b[...] + d1b[...] + 1.0
    dis = lax.rsqrt(deg)
    disb[...] = dis
    h = jnp.dot(xb[...], w1b[...], preferred_element_type=F32)
    g1b[...] = h * dis[:, None]


def _tc1(x_pad, w1, deg0, deg1):
    return pl.pallas_call(
        _tc1_body,
        grid=(10,),
        in_specs=[
            pl.BlockSpec((NB, D), lambda i: (i, 0)),
            pl.BlockSpec((D, H), lambda i: (0, 0)),
            pl.BlockSpec((NB,), lambda i: (i,)),
            pl.BlockSpec((NB,), lambda i: (i,)),
        ],
        out_specs=[
            pl.BlockSpec((NB, H), lambda i: (i, 0)),
            pl.BlockSpec((NB,), lambda i: (i,)),
        ],
        out_shape=[
            jax.ShapeDtypeStruct((NPAD, H), F32),
            jax.ShapeDtypeStruct((NPAD,), F32),
        ],
    )(x_pad, w1, deg0, deg1)


def _tc2_body(a0b, a1b, g1b, db, b1b, w2b, g2b):
    dis = db[...]
    t = dis[:, None] * (a0b[...] + a1b[...] + g1b[...]) + b1b[0:1, :]
    t = jnp.maximum(t, 0.0)
    g2b[...] = jnp.dot(t, w2b[...], preferred_element_type=F32) * dis[:, None]


def _tc2(a0, a1, g1, dis, b1b, w2):
    return pl.pallas_call(
        _tc2_body,
        grid=(10,),
        in_specs=[
            pl.BlockSpec((NB, H), lambda i: (i, 0)),
            pl.BlockSpec((NB, H), lambda i: (i, 0)),
            pl.BlockSpec((NB, H), lambda i: (i, 0)),
            pl.BlockSpec((NB,), lambda i: (i,)),
            pl.BlockSpec((8, H), lambda i: (0, 0)),
            pl.BlockSpec((H, H), lambda i: (0, 0)),
        ],
        out_specs=pl.BlockSpec((NB, H), lambda i: (i, 0)),
        out_shape=jax.ShapeDtypeStruct((NPAD, H), F32),
    )(a0, a1, g1, dis, b1b, w2)


def _tc3_body(a0b, a1b, g2b, db, b2b, wabb, atb, wmb, bpb, abb, eab):
    dis = db[...]
    t = dis[:, None] * (a0b[...] + a1b[...] + g2b[...]) + b2b[0:1, :]
    t = jnp.maximum(t, 0.0)
    abb[...] = jnp.dot(t, wabb[...], preferred_element_type=F32)
    eab[...] = jnp.dot(atb[...], wmb[...], preferred_element_type=F32) + bpb[0:1, :]


def _tc3(a0, a1, g2, dis, b2b, wab, attr2d, wmat, bpb):
    return pl.pallas_call(
        _tc3_body,
        grid=(10,),
        in_specs=[
            pl.BlockSpec((NB, H), lambda i: (i, 0)),
            pl.BlockSpec((NB, H), lambda i: (i, 0)),
            pl.BlockSpec((NB, H), lambda i: (i, 0)),
            pl.BlockSpec((NB,), lambda i: (i,)),
            pl.BlockSpec((8, H), lambda i: (0, 0)),
            pl.BlockSpec((H, 128), lambda i: (0, 0)),
            pl.BlockSpec((AR, 128), lambda i: (i, 0)),
            pl.BlockSpec((128, 128), lambda i: (0, 0)),
            pl.BlockSpec((8, 128), lambda i: (0, 0)),
        ],
        out_specs=[
            pl.BlockSpec((NB, 128), lambda i: (i, 0)),
            pl.BlockSpec((AR, 128), lambda i: (i, 0)),
        ],
        out_shape=[
            jax.ShapeDtypeStruct((NPAD, 128), F32),
            jax.ShapeDtypeStruct((N, 128), F32),
        ],
    )(a0, a1, g2, dis, b2b, wab, attr2d, wmat, bpb)


def kernel(x, edge_index, edge_attr, W1, b1, W2, b2, Wp, bp):
    ei = edge_index.astype(jnp.int32)
    src = ei[0]
    dst = ei[1]
    src3 = src.reshape(NW, NCHUNK, CH)
    dst3 = dst.reshape(NW, NCHUNK, CH)

    x_pad = jnp.pad(x, ((0, NPAD - N), (0, 0)))
    attr2d = edge_attr.reshape(N, 128)

    wp_e = Wp[2 * H:, 0]
    eye32 = jnp.eye(32, dtype=F32)
    wmat = jnp.pad(jnp.kron(eye32, wp_e[:, None]), ((0, 0), (0, 96)))

    wab = jnp.zeros((H, 128), F32)
    wab = wab.at[:, 0].set(Wp[:H, 0]).at[:, 1].set(Wp[H:2 * H, 0])

    bpb = jnp.broadcast_to(bp.reshape(1, 1), (8, 128)).astype(F32)
    b1b = jnp.broadcast_to(b1[None, :], (8, H))
    b2b = jnp.broadcast_to(b2[None, :], (8, H))

    degp = _degree(dst3)
    g1, dis = _tc1(x_pad, W1, degp[0], degp[1])
    a10, a11 = _scatter_rows(g1, src, dst3)
    g2 = _tc2(a10, a11, g1, dis, b1b, W2)
    a20, a21 = _scatter_rows(g2, src, dst3)
    abm, ea2d = _tc3(a20, a21, g2, dis, b2b, wab, attr2d, wmat, bpb)
    ea = ea2d[:, :32].reshape(E)
    scores = _score(abm[:, 0], abm[:, 1], ea, src, dst)
    return scores.reshape(E, 1)


# speedup vs baseline: 28.7030x; 1.1940x over previous
import jax
import jax.numpy as jnp
from jax import lax
from jax.experimental import pallas as pl
from jax.experimental.pallas import tpu as pltpu
from jax.experimental.pallas import tpu_sc as plsc

N = 10000
E = 320000
D = 128
H = 128

NC = 2
NS = 16
NW = NC * NS
EW = E // NW
CH = 80
NCHUNK = EW // CH
NPAD = 10240
RW = NPAD // NS

F32 = jnp.float32


def _fill_1d(ref, n, val):
    v = jnp.full((16,), val, F32)

    def body(i, _):
        ref[pl.ds(i * 16, 16)] = v
        return 0

    lax.fori_loop(0, n // 16, body, 0)


def _fill_zero_2d(ref, rows):
    zv = jnp.zeros((16,), F32)

    def body(i, _):
        for j in range(D // 16):
            ref[i, pl.ds(j * 16, 16)] = zv
        return 0

    lax.fori_loop(0, rows, body, 0)


def _sc_degree(dst3, out, cnt, idx2, ones_v, zb):
    c = lax.axis_index("c")
    s = lax.axis_index("s")
    wid = s * NC + c

    _fill_1d(zb, RW, 0.0)
    _fill_1d(ones_v, CH, 1.0)

    pltpu.sync_copy(dst3.at[wid], idx2)
    pltpu.sync_copy(zb, cnt.at[pl.ds(s * RW, RW)])
    plsc.subcore_barrier()

    def body(k, _):
        pltpu.sync_copy(ones_v, cnt.at[idx2.at[k]], add=True)
        return 0

    lax.fori_loop(0, NCHUNK, body, 0)
    plsc.subcore_barrier()
    pltpu.sync_copy(cnt.at[pl.ds(s * RW, RW)], out.at[c, pl.ds(s * RW, RW)])


def _degree(dst3):
    return pl.kernel(
        _sc_degree,
        out_type=jax.ShapeDtypeStruct((NC, NPAD), F32),
        mesh=plsc.VectorSubcoreMesh(core_axis_name="c", subcore_axis_name="s"),
        scratch_types=[
            pltpu.VMEM_SHARED((NPAD,), F32),
            pltpu.VMEM((NCHUNK, CH), jnp.int32),
            pltpu.VMEM((CH,), F32),
            pltpu.VMEM((RW,), F32),
        ],
    )(dst3)


def _sc_scatter(g, src, dst3, out0, out1, acc, src1, dst2, rows_a, rows_b,
                sem_a, sem_b):
    c = lax.axis_index("c")
    s = lax.axis_index("s")
    wid = s * NC + c

    _fill_zero_2d(rows_a, CH)
    pltpu.sync_copy(src.at[pl.ds(wid * EW, EW)], src1)
    pltpu.sync_copy(dst3.at[wid], dst2)
    for q in range(RW // CH):
        pltpu.sync_copy(rows_a, acc.at[pl.ds(s * RW + q * CH, CH)])
    plsc.subcore_barrier()

    def idx(k):
        return src1.at[pl.ds(k * CH, CH)]

    pltpu.async_copy(g.at[idx(0)], rows_a, sem_a)
    pltpu.async_copy(g.at[idx(1)], rows_b, sem_b)

    def body(i, _):
        k0 = 2 * i
        pltpu.make_async_copy(g.at[idx(k0)], rows_a, sem_a).wait()
        pltpu.sync_copy(rows_a, acc.at[dst2.at[k0]], add=True)
        pltpu.async_copy(g.at[idx(k0 + 2)], rows_a, sem_a)
        pltpu.make_async_copy(g.at[idx(k0 + 1)], rows_b, sem_b).wait()
        pltpu.sync_copy(rows_b, acc.at[dst2.at[k0 + 1]], add=True)

        @pl.when(i < (NCHUNK - 3) // 2)
        def _():
            pltpu.async_copy(g.at[idx(k0 + 3)], rows_b, sem_b)

        return 0

    lax.fori_loop(0, (NCHUNK - 1) // 2, body, 0)
    pltpu.make_async_copy(g.at[idx(NCHUNK - 1)], rows_a, sem_a).wait()
    pltpu.sync_copy(rows_a, acc.at[dst2.at[NCHUNK - 1]], add=True)

    plsc.subcore_barrier()

    @pl.when(c == 0)
    def _():
        pltpu.sync_copy(acc.at[pl.ds(s * RW, RW)], out0.at[pl.ds(s * RW, RW)])

    @pl.when(c == 1)
    def _():
        pltpu.sync_copy(acc.at[pl.ds(s * RW, RW)], out1.at[pl.ds(s * RW, RW)])


def _scatter_rows(g, src, dst3):
    return pl.kernel(
        _sc_scatter,
        out_type=(jax.ShapeDtypeStruct((NPAD, D), F32),
                  jax.ShapeDtypeStruct((NPAD, D), F32)),
        mesh=plsc.VectorSubcoreMesh(core_axis_name="c", subcore_axis_name="s"),
        scratch_types=[
            pltpu.VMEM_SHARED((NPAD, D), F32),
            pltpu.VMEM((EW,), jnp.int32),
            pltpu.VMEM((NCHUNK, CH), jnp.int32),
            pltpu.VMEM((CH, D), F32),
            pltpu.VMEM((CH, D), F32),
            pltpu.SemaphoreType.DMA,
            pltpu.SemaphoreType.DMA,
        ],
    )(g, src, dst3)


def _sc_score(a, b, ea, src, dst, out, a_v, b_v, sv, dv, ev, ov):
    c = lax.axis_index("c")
    s = lax.axis_index("s")
    wid = s * NC + c
    base = wid * EW

    pltpu.sync_copy(a, a_v)
    pltpu.sync_copy(b, b_v)
    pltpu.sync_copy(src.at[pl.ds(base, EW)], sv)
    pltpu.sync_copy(dst.at[pl.ds(base, EW)], dv)
    pltpu.sync_copy(ea.at[pl.ds(base, EW)], ev)

    def body(j, _):
        o = j * 16
        s16 = sv[pl.ds(o, 16)]
        d16 = dv[pl.ds(o, 16)]
        va = plsc.load_gather(a_v, [s16])
        vb = plsc.load_gather(b_v, [d16])
        ov[pl.ds(o, 16)] = va + vb + ev[pl.ds(o, 16)]
        return 0

    lax.fori_loop(0, EW // 16, body, 0)
    pltpu.sync_copy(ov, out.at[pl.ds(base, EW)])


def _score(a, b, ea, src, dst):
    return pl.kernel(
        _sc_score,
        out_type=jax.ShapeDtypeStruct((E,), F32),
        mesh=plsc.VectorSubcoreMesh(core_axis_name="c", subcore_axis_name="s"),
        compiler_params=pltpu.CompilerParams(needs_layout_passes=False),
        scratch_types=[
            pltpu.VMEM((NPAD,), F32),
            pltpu.VMEM((NPAD,), F32),
            pltpu.VMEM((EW,), jnp.int32),
            pltpu.VMEM((EW,), jnp.int32),
            pltpu.VMEM((EW,), F32),
            pltpu.VMEM((EW,), F32),
        ],
    )(a, b, ea, src, dst)


NB = NPAD // 10
AR = N // 10


def _tc1_body(xb, w1b, d0b, d1b, g1b, disb):
    deg = d0b[...] + d1b[...] + 1.0
    dis = lax.rsqrt(deg)
    disb[...] = dis
    h = jnp.dot(xb[...], w1b[...], preferred_element_type=F32)
    g1b[...] = h * dis[:, None]


def _tc1(x_pad, w1, deg0, deg1):
    return pl.pallas_call(
        _tc1_body,
        grid=(10,),
        in_specs=[
            pl.BlockSpec((NB, D), lambda i: (i, 0)),
            pl.BlockSpec((D, H), lambda i: (0, 0)),
            pl.BlockSpec((NB,), lambda i: (i,)),
            pl.BlockSpec((NB,), lambda i: (i,)),
        ],
        out_specs=[
            pl.BlockSpec((NB, H), lambda i: (i, 0)),
            pl.BlockSpec((NB,), lambda i: (i,)),
        ],
        out_shape=[
            jax.ShapeDtypeStruct((NPAD, H), F32),
            jax.ShapeDtypeStruct((NPAD,), F32),
        ],
    )(x_pad, w1, deg0, deg1)


AR_A = 5120 // 10
AR_B = 4880 // 10


def _tc2_body(a0b, a1b, g1b, db, b1b, w2b, atb, wmb, bpb, g2b, eab):
    dis = db[...]
    t = dis[:, None] * (a0b[...] + a1b[...] + g1b[...]) + b1b[0:1, :]
    t = jnp.maximum(t, 0.0)
    g2b[...] = jnp.dot(t, w2b[...], preferred_element_type=F32) * dis[:, None]
    eab[...] = jnp.dot(atb[...], wmb[...], preferred_element_type=F32) + bpb[0:1, :]


def _tc2(a0, a1, g1, dis, b1b, w2, attr_a, wmat, bpb):
    return pl.pallas_call(
        _tc2_body,
        grid=(10,),
        in_specs=[
            pl.BlockSpec((NB, H), lambda i: (i, 0)),
            pl.BlockSpec((NB, H), lambda i: (i, 0)),
            pl.BlockSpec((NB, H), lambda i: (i, 0)),
            pl.BlockSpec((NB,), lambda i: (i,)),
            pl.BlockSpec((8, H), lambda i: (0, 0)),
            pl.BlockSpec((H, H), lambda i: (0, 0)),
            pl.BlockSpec((AR_A, 128), lambda i: (i, 0)),
            pl.BlockSpec((128, 128), lambda i: (0, 0)),
            pl.BlockSpec((8, 128), lambda i: (0, 0)),
        ],
        out_specs=[
            pl.BlockSpec((NB, H), lambda i: (i, 0)),
            pl.BlockSpec((AR_A, 128), lambda i: (i, 0)),
        ],
        out_shape=[
            jax.ShapeDtypeStruct((NPAD, H), F32),
            jax.ShapeDtypeStruct((5120, 128), F32),
        ],
    )(a0, a1, g1, dis, b1b, w2, attr_a, wmat, bpb)


def _tc3_body(a0b, a1b, g2b, db, b2b, wabb, atb, wmb, bpb, abb, eab):
    dis = db[...]
    t = dis[:, None] * (a0b[...] + a1b[...] + g2b[...]) + b2b[0:1, :]
    t = jnp.maximum(t, 0.0)
    abb[...] = jnp.dot(t, wabb[...], preferred_element_type=F32)
    eab[...] = jnp.dot(atb[...], wmb[...], preferred_element_type=F32) + bpb[0:1, :]


def _tc3(a0, a1, g2, dis, b2b, wab, attr_b, wmat, bpb):
    return pl.pallas_call(
        _tc3_body,
        grid=(10,),
        in_specs=[
            pl.BlockSpec((NB, H), lambda i: (i, 0)),
            pl.BlockSpec((NB, H), lambda i: (i, 0)),
            pl.BlockSpec((NB, H), lambda i: (i, 0)),
            pl.BlockSpec((NB,), lambda i: (i,)),
            pl.BlockSpec((8, H), lambda i: (0, 0)),
            pl.BlockSpec((H, 128), lambda i: (0, 0)),
            pl.BlockSpec((AR_B, 128), lambda i: (i, 0)),
            pl.BlockSpec((128, 128), lambda i: (0, 0)),
            pl.BlockSpec((8, 128), lambda i: (0, 0)),
        ],
        out_specs=[
            pl.BlockSpec((NB, 128), lambda i: (i, 0)),
            pl.BlockSpec((AR_B, 128), lambda i: (i, 0)),
        ],
        out_shape=[
            jax.ShapeDtypeStruct((NPAD, 128), F32),
            jax.ShapeDtypeStruct((4880, 128), F32),
        ],
    )(a0, a1, g2, dis, b2b, wab, attr_b, wmat, bpb)


def kernel(x, edge_index, edge_attr, W1, b1, W2, b2, Wp, bp):
    ei = edge_index.astype(jnp.int32)
    src = ei[0]
    dst = ei[1]
    src3 = src.reshape(NW, NCHUNK, CH)
    dst3 = dst.reshape(NW, NCHUNK, CH)

    x_pad = jnp.pad(x, ((0, NPAD - N), (0, 0)))
    attr_a = edge_attr[:163840].reshape(5120, 128)
    attr_b = edge_attr[163840:].reshape(4880, 128)

    wp_e = Wp[2 * H:, 0]
    eye32 = jnp.eye(32, dtype=F32)
    wmat = jnp.pad(jnp.kron(eye32, wp_e[:, None]), ((0, 0), (0, 96)))

    wab = jnp.zeros((H, 128), F32)
    wab = wab.at[:, 0].set(Wp[:H, 0]).at[:, 1].set(Wp[H:2 * H, 0])

    bpb = jnp.broadcast_to(bp.reshape(1, 1), (8, 128)).astype(F32)
    b1b = jnp.broadcast_to(b1[None, :], (8, H))
    b2b = jnp.broadcast_to(b2[None, :], (8, H))

    degp = _degree(dst3)
    g1, dis = _tc1(x_pad, W1, degp[0], degp[1])
    a10, a11 = _scatter_rows(g1, src, dst3)
    g2, ea_a = _tc2(a10, a11, g1, dis, b1b, W2, attr_a, wmat, bpb)
    a20, a21 = _scatter_rows(g2, src, dst3)
    abm, ea_b = _tc3(a20, a21, g2, dis, b2b, wab, attr_b, wmat, bpb)
    ea = jnp.concatenate([ea_a[:, :32].reshape(163840),
                          ea_b[:, :32].reshape(156160)])
    scores = _score(abm[:, 0], abm[:, 1], ea, src, dst)
    return scores.reshape(E, 1)


# speedup vs baseline: 29.2739x; 1.0199x over previous
import jax
import jax.numpy as jnp
from jax import lax
from jax.experimental import pallas as pl
from jax.experimental.pallas import tpu as pltpu
from jax.experimental.pallas import tpu_sc as plsc

N = 10000
E = 320000
D = 128
H = 128

NC = 2
NS = 16
NW = NC * NS
EW = E // NW
CH = 80
NCHUNK = EW // CH
NPAD = 10240
RW = NPAD // NS

F32 = jnp.float32


def _fill_1d(ref, n, val):
    v = jnp.full((16,), val, F32)

    def body(i, _):
        ref[pl.ds(i * 16, 16)] = v
        return 0

    lax.fori_loop(0, n // 16, body, 0)


def _fill_zero_2d(ref, rows):
    zv = jnp.zeros((16,), F32)

    def body(i, _):
        for j in range(D // 16):
            ref[i, pl.ds(j * 16, 16)] = zv
        return 0

    lax.fori_loop(0, rows, body, 0)


def _sc_degree(dst3, out, cnt, idx2, ones_v, zb):
    c = lax.axis_index("c")
    s = lax.axis_index("s")
    wid = s * NC + c

    _fill_1d(zb, RW, 0.0)
    _fill_1d(ones_v, CH, 1.0)

    pltpu.sync_copy(dst3.at[wid], idx2)
    pltpu.sync_copy(zb, cnt.at[pl.ds(s * RW, RW)])
    plsc.subcore_barrier()

    def body(k, _):
        pltpu.sync_copy(ones_v, cnt.at[idx2.at[k]], add=True)
        return 0

    lax.fori_loop(0, NCHUNK, body, 0)
    plsc.subcore_barrier()
    pltpu.sync_copy(cnt.at[pl.ds(s * RW, RW)], out.at[c, pl.ds(s * RW, RW)])


def _degree(dst3):
    return pl.kernel(
        _sc_degree,
        out_type=jax.ShapeDtypeStruct((NC, NPAD), F32),
        mesh=plsc.VectorSubcoreMesh(core_axis_name="c", subcore_axis_name="s"),
        scratch_types=[
            pltpu.VMEM_SHARED((NPAD,), F32),
            pltpu.VMEM((NCHUNK, CH), jnp.int32),
            pltpu.VMEM((CH,), F32),
            pltpu.VMEM((RW,), F32),
        ],
    )(dst3)


def _sc_scatter(g, src, dst3, out0, out1, acc, src1, dst2, rows_a, rows_b,
                sem_a, sem_b):
    c = lax.axis_index("c")
    s = lax.axis_index("s")
    wid = s * NC + c

    _fill_zero_2d(rows_a, CH)
    pltpu.sync_copy(src.at[pl.ds(wid * EW, EW)], src1)
    pltpu.sync_copy(dst3.at[wid], dst2)
    for q in range(RW // CH):
        pltpu.sync_copy(rows_a, acc.at[pl.ds(s * RW + q * CH, CH)])
    plsc.subcore_barrier()

    def idx(k):
        return src1.at[pl.ds(k * CH, CH)]

    pltpu.async_copy(g.at[idx(0)], rows_a, sem_a)
    pltpu.async_copy(g.at[idx(1)], rows_b, sem_b)

    def body(i, _):
        k0 = 2 * i
        pltpu.make_async_copy(g.at[idx(k0)], rows_a, sem_a).wait()
        pltpu.sync_copy(rows_a, acc.at[dst2.at[k0]], add=True)
        pltpu.async_copy(g.at[idx(k0 + 2)], rows_a, sem_a)
        pltpu.make_async_copy(g.at[idx(k0 + 1)], rows_b, sem_b).wait()
        pltpu.sync_copy(rows_b, acc.at[dst2.at[k0 + 1]], add=True)

        @pl.when(i < (NCHUNK - 3) // 2)
        def _():
            pltpu.async_copy(g.at[idx(k0 + 3)], rows_b, sem_b)

        return 0

    lax.fori_loop(0, (NCHUNK - 1) // 2, body, 0)
    pltpu.make_async_copy(g.at[idx(NCHUNK - 1)], rows_a, sem_a).wait()
    pltpu.sync_copy(rows_a, acc.at[dst2.at[NCHUNK - 1]], add=True)

    plsc.subcore_barrier()

    @pl.when(c == 0)
    def _():
        pltpu.sync_copy(acc.at[pl.ds(s * RW, RW)], out0.at[pl.ds(s * RW, RW)])

    @pl.when(c == 1)
    def _():
        pltpu.sync_copy(acc.at[pl.ds(s * RW, RW)], out1.at[pl.ds(s * RW, RW)])


def _scatter_rows(g, src, dst3):
    return pl.kernel(
        _sc_scatter,
        out_type=(jax.ShapeDtypeStruct((NPAD, D), F32),
                  jax.ShapeDtypeStruct((NPAD, D), F32)),
        mesh=plsc.VectorSubcoreMesh(core_axis_name="c", subcore_axis_name="s"),
        scratch_types=[
            pltpu.VMEM_SHARED((NPAD, D), F32),
            pltpu.VMEM((EW,), jnp.int32),
            pltpu.VMEM((NCHUNK, CH), jnp.int32),
            pltpu.VMEM((CH, D), F32),
            pltpu.VMEM((CH, D), F32),
            pltpu.SemaphoreType.DMA,
            pltpu.SemaphoreType.DMA,
        ],
    )(g, src, dst3)


def _sc_score(a, b, ea, src, dst, out, a_v, b_v, sv, dv, ev, ov):
    c = lax.axis_index("c")
    s = lax.axis_index("s")
    wid = s * NC + c
    base = wid * EW

    pltpu.sync_copy(a, a_v)
    pltpu.sync_copy(b, b_v)
    pltpu.sync_copy(src.at[pl.ds(base, EW)], sv)
    pltpu.sync_copy(dst.at[pl.ds(base, EW)], dv)
    pltpu.sync_copy(ea.at[pl.ds(base, EW)], ev)

    def body(j, _):
        o = j * 16
        s16 = sv[pl.ds(o, 16)]
        d16 = dv[pl.ds(o, 16)]
        va = plsc.load_gather(a_v, [s16])
        vb = plsc.load_gather(b_v, [d16])
        ov[pl.ds(o, 16)] = va + vb + ev[pl.ds(o, 16)]
        return 0

    lax.fori_loop(0, EW // 16, body, 0)
    pltpu.sync_copy(ov, out.at[pl.ds(base, EW)])


def _score(a, b, ea, src, dst):
    return pl.kernel(
        _sc_score,
        out_type=jax.ShapeDtypeStruct((E,), F32),
        mesh=plsc.VectorSubcoreMesh(core_axis_name="c", subcore_axis_name="s"),
        compiler_params=pltpu.CompilerParams(needs_layout_passes=False),
        scratch_types=[
            pltpu.VMEM((NPAD,), F32),
            pltpu.VMEM((NPAD,), F32),
            pltpu.VMEM((EW,), jnp.int32),
            pltpu.VMEM((EW,), jnp.int32),
            pltpu.VMEM((EW,), F32),
            pltpu.VMEM((EW,), F32),
        ],
    )(a, b, ea, src, dst)


NB = NPAD // 10
AR = N // 10


def _tc1_body(xb, w1b, d0b, d1b, g1b, disb):
    deg = d0b[...] + d1b[...] + 1.0
    dis = lax.rsqrt(deg)
    disb[...] = dis
    h = jnp.dot(xb[...], w1b[...], preferred_element_type=F32)
    g1b[...] = h * dis[:, None]


def _tc1(x_pad, w1, deg0, deg1):
    return pl.pallas_call(
        _tc1_body,
        grid=(10,),
        in_specs=[
            pl.BlockSpec((NB, D), lambda i: (i, 0)),
            pl.BlockSpec((D, H), lambda i: (0, 0)),
            pl.BlockSpec((NB,), lambda i: (i,)),
            pl.BlockSpec((NB,), lambda i: (i,)),
        ],
        out_specs=[
            pl.BlockSpec((NB, H), lambda i: (i, 0)),
            pl.BlockSpec((NB,), lambda i: (i,)),
        ],
        out_shape=[
            jax.ShapeDtypeStruct((NPAD, H), F32),
            jax.ShapeDtypeStruct((NPAD,), F32),
        ],
    )(x_pad, w1, deg0, deg1)


AR_A = 5120 // 10
AR_B = 4880 // 10


def _tc2_body(a0b, a1b, g1b, db, b1b, w2b, atb, wmb, bpb, g2b, eab):
    dis = db[...]
    t = dis[:, None] * (a0b[...] + a1b[...] + g1b[...]) + b1b[0:1, :]
    t = jnp.maximum(t, 0.0)
    g2b[...] = jnp.dot(t, w2b[...], preferred_element_type=F32) * dis[:, None]
    eab[...] = jnp.dot(atb[...], wmb[...], preferred_element_type=F32) + bpb[0:1, :]


def _tc2(a0, a1, g1, dis, b1b, w2, attr_a, wmat, bpb):
    return pl.pallas_call(
        _tc2_body,
        grid=(10,),
        in_specs=[
            pl.BlockSpec((NB, H), lambda i: (i, 0)),
            pl.BlockSpec((NB, H), lambda i: (i, 0)),
            pl.BlockSpec((NB, H), lambda i: (i, 0)),
            pl.BlockSpec((NB,), lambda i: (i,)),
            pl.BlockSpec((8, H), lambda i: (0, 0)),
            pl.BlockSpec((H, H), lambda i: (0, 0)),
            pl.BlockSpec((AR_A, 128), lambda i: (i, 0)),
            pl.BlockSpec((128, 128), lambda i: (0, 0)),
            pl.BlockSpec((8, 128), lambda i: (0, 0)),
        ],
        out_specs=[
            pl.BlockSpec((NB, H), lambda i: (i, 0)),
            pl.BlockSpec((AR_A, 128), lambda i: (i, 0)),
        ],
        out_shape=[
            jax.ShapeDtypeStruct((NPAD, H), F32),
            jax.ShapeDtypeStruct((5120, 128), F32),
        ],
    )(a0, a1, g1, dis, b1b, w2, attr_a, wmat, bpb)


def _tc3_body(a0b, a1b, g2b, db, b2b, wabb, atb, wmb, bpb, avb, bvb, eab):
    dis = db[...]
    t = dis[:, None] * (a0b[...] + a1b[...] + g2b[...]) + b2b[0:1, :]
    t = jnp.maximum(t, 0.0)
    ab = jnp.dot(t, wabb[...], preferred_element_type=F32)
    avb[...] = ab[:, 0]
    bvb[...] = ab[:, 1]
    eab[...] = jnp.dot(atb[...], wmb[...], preferred_element_type=F32) + bpb[0:1, :]


def _tc3(a0, a1, g2, dis, b2b, wab, attr_b, wmat, bpb):
    return pl.pallas_call(
        _tc3_body,
        grid=(10,),
        in_specs=[
            pl.BlockSpec((NB, H), lambda i: (i, 0)),
            pl.BlockSpec((NB, H), lambda i: (i, 0)),
            pl.BlockSpec((NB, H), lambda i: (i, 0)),
            pl.BlockSpec((NB,), lambda i: (i,)),
            pl.BlockSpec((8, H), lambda i: (0, 0)),
            pl.BlockSpec((H, 128), lambda i: (0, 0)),
            pl.BlockSpec((AR_B, 128), lambda i: (i, 0)),
            pl.BlockSpec((128, 128), lambda i: (0, 0)),
            pl.BlockSpec((8, 128), lambda i: (0, 0)),
        ],
        out_specs=[
            pl.BlockSpec((NB,), lambda i: (i,)),
            pl.BlockSpec((NB,), lambda i: (i,)),
            pl.BlockSpec((AR_B, 128), lambda i: (i, 0)),
        ],
        out_shape=[
            jax.ShapeDtypeStruct((NPAD,), F32),
            jax.ShapeDtypeStruct((NPAD,), F32),
            jax.ShapeDtypeStruct((4880, 128), F32),
        ],
    )(a0, a1, g2, dis, b2b, wab, attr_b, wmat, bpb)


def kernel(x, edge_index, edge_attr, W1, b1, W2, b2, Wp, bp):
    ei = edge_index.astype(jnp.int32)
    src = ei[0]
    dst = ei[1]
    src3 = src.reshape(NW, NCHUNK, CH)
    dst3 = dst.reshape(NW, NCHUNK, CH)

    x_pad = jnp.pad(x, ((0, NPAD - N), (0, 0)))
    attr_a = edge_attr[:163840].reshape(5120, 128)
    attr_b = edge_attr[163840:].reshape(4880, 128)

    wp_e = Wp[2 * H:, 0]
    eye32 = jnp.eye(32, dtype=F32)
    wmat = jnp.pad(jnp.kron(eye32, wp_e[:, None]), ((0, 0), (0, 96)))

    wab = jnp.zeros((H, 128), F32)
    wab = wab.at[:, 0].set(Wp[:H, 0]).at[:, 1].set(Wp[H:2 * H, 0])

    bpb = jnp.broadcast_to(bp.reshape(1, 1), (8, 128)).astype(F32)
    b1b = jnp.broadcast_to(b1[None, :], (8, H))
    b2b = jnp.broadcast_to(b2[None, :], (8, H))

    degp = _degree(dst3)
    g1, dis = _tc1(x_pad, W1, degp[0], degp[1])
    a10, a11 = _scatter_rows(g1, src, dst3)
    g2, ea_a = _tc2(a10, a11, g1, dis, b1b, W2, attr_a, wmat, bpb)
    a20, a21 = _scatter_rows(g2, src, dst3)
    av, bv, ea_b = _tc3(a20, a21, g2, dis, b2b, wab, attr_b, wmat, bpb)
    ea = jnp.concatenate([ea_a[:, :32].reshape(163840),
                          ea_b[:, :32].reshape(156160)])
    scores = _score(av, bv, ea, src, dst)
    return scores.reshape(E, 1)
